# Initial kernel scaffold; baseline (speedup 1.0000x reference)
import jax
import jax.numpy as jnp
from jax.experimental import pallas as pl


def kernel(x, edge_index, enc_Ws, enc_bs, enc_gs, enc_betas, node_Ws, node_bs, node_gs, node_betas, edge_Ws, edge_bs, edge_gs, edge_betas):
    raise NotImplementedError("write your pallas kernel here")


# speedup vs baseline: 3.0246x; 3.0246x over previous
import functools

import jax
import jax.numpy as jnp
from jax import lax
from jax.experimental import pallas as pl
from jax.experimental.pallas import tpu as pltpu
from jax.experimental.pallas import tpu_sc as plsc

N = 10000
E = 320000
H = 64
C_IN = 128
ITERS = 8
EPS = 1e-5

RB = 256
N_PAD = 10240
NBLK = N_PAD // RB

NC, NS = 2, 16
NW = NC * NS
CHUNK = 128

MSG_CHUNKS = 158
MSG_EPT = MSG_CHUNKS * CHUNK
MSG_PAD = NW * MSG_EPT

EG_CHUNKS = 80
EG_EPT = EG_CHUNKS * CHUNK
EG_PAD = NW * EG_EPT
EB = 512
EBLK = EG_PAD // EB

DUMMY = N_PAD - 1
ZROWS = N_PAD // NS


def _dot(a, b):
    return lax.dot_general(a.astype(jnp.bfloat16), b.astype(jnp.bfloat16),
                           (((1,), (0,)), ((), ())),
                           preferred_element_type=jnp.float32)


def _ln_relu(t, g, b):
    mu = jnp.mean(t, axis=-1, keepdims=True)
    var = jnp.mean((t - mu) ** 2, axis=-1, keepdims=True)
    t = (t - mu) * lax.rsqrt(var + EPS)
    return jnp.maximum(t * g + b, 0.0)


def _enc_body(x_ref, w1, b1, g1, be1, w2, b2, g2, be2, w3, b3, g3, be3, o_ref):
    t = _ln_relu(_dot(x_ref[...], w1[...]) + b1[...], g1[...], be1[...])
    t = _ln_relu(_dot(t, w2[...]) + b2[...], g2[...], be2[...])
    o_ref[...] = _ln_relu(_dot(t, w3[...]) + b3[...], g3[...], be3[...])


def _node_body(h_ref, m0_ref, m1_ref, w1a, w1b, b1, g1, be1,
               w2, b2, g2, be2, w3, b3, g3, be3, o_ref):
    m = m0_ref[...] + m1_ref[...]
    t = _dot(h_ref[...], w1a[...]) + _dot(m, w1b[...]) + b1[...]
    t = _ln_relu(t, g1[...], be1[...])
    t = _ln_relu(_dot(t, w2[...]) + b2[...], g2[...], be2[...])
    o_ref[...] = _ln_relu(_dot(t, w3[...]) + b3[...], g3[...], be3[...])


def _ab_body(h_ref, wa, wb, a_ref, b_ref):
    a_ref[...] = _dot(h_ref[...], wa[...])
    b_ref[...] = _dot(h_ref[...], wb[...])


def _edge_body(u_ref, v_ref, b1, g1, be1, w2, b2, g2, be2,
               w3, b3, g3, be3, w4, b4, o_ref):
    t = _ln_relu(u_ref[...] + v_ref[...] + b1[...], g1[...], be1[...])
    t = _ln_relu(_dot(t, w2[...]) + b2[...], g2[...], be2[...])
    t = _ln_relu(_dot(t, w3[...]) + b3[...], g3[...], be3[...])
    o_ref[...] = _dot(t, w4[...]) + b4[...]


def _wspec(r, c):
    return pl.BlockSpec((r, c), lambda i: (0, 0))


_enc = pl.pallas_call(
    _enc_body,
    grid=(NBLK,),
    in_specs=[pl.BlockSpec((RB, C_IN), lambda i: (i, 0)),
              _wspec(C_IN, H), _wspec(1, H), _wspec(1, H), _wspec(1, H),
              _wspec(H, H), _wspec(1, H), _wspec(1, H), _wspec(1, H),
              _wspec(H, H), _wspec(1, H), _wspec(1, H), _wspec(1, H)],
    out_specs=pl.BlockSpec((RB, H), lambda i: (i, 0)),
    out_shape=jax.ShapeDtypeStruct((N_PAD, H), jnp.float32),
)

_node = pl.pallas_call(
    _node_body,
    grid=(NBLK,),
    in_specs=[pl.BlockSpec((RB, H), lambda i: (i, 0)),
              pl.BlockSpec((RB, H), lambda i: (i, 0)),
              pl.BlockSpec((RB, H), lambda i: (i, 0)),
              _wspec(H, H), _wspec(H, H), _wspec(1, H), _wspec(1, H), _wspec(1, H),
              _wspec(H, H), _wspec(1, H), _wspec(1, H), _wspec(1, H),
              _wspec(H, H), _wspec(1, H), _wspec(1, H), _wspec(1, H)],
    out_specs=pl.BlockSpec((RB, H), lambda i: (i, 0)),
    out_shape=jax.ShapeDtypeStruct((N_PAD, H), jnp.float32),
)

_ab = pl.pallas_call(
    _ab_body,
    grid=(NBLK,),
    in_specs=[pl.BlockSpec((RB, H), lambda i: (i, 0)),
              _wspec(H, H), _wspec(H, H)],
    out_specs=[pl.BlockSpec((RB, H), lambda i: (i, 0)),
               pl.BlockSpec((RB, H), lambda i: (i, 0))],
    out_shape=[jax.ShapeDtypeStruct((N_PAD, H), jnp.float32),
               jax.ShapeDtypeStruct((N_PAD, H), jnp.float32)],
)

_edge = pl.pallas_call(
    _edge_body,
    grid=(EBLK,),
    in_specs=[pl.BlockSpec((EB, H), lambda i: (i, 0)),
              pl.BlockSpec((EB, H), lambda i: (i, 0)),
              _wspec(1, H), _wspec(1, H), _wspec(1, H),
              _wspec(H, H), _wspec(1, H), _wspec(1, H), _wspec(1, H),
              _wspec(H, H), _wspec(1, H), _wspec(1, H), _wspec(1, H),
              _wspec(H, 1), _wspec(1, 1)],
    out_specs=pl.BlockSpec((EB, 1), lambda i: (i, 0)),
    out_shape=jax.ShapeDtypeStruct((EG_PAD, 1), jnp.float32),
)


@functools.cache
def _sc_kernels():
    mesh = plsc.VectorSubcoreMesh(core_axis_name="c", subcore_axis_name="s",
                                  num_cores=NC, num_subcores=NS)

    @functools.partial(
        pl.kernel,
        out_type=jax.ShapeDtypeStruct((NC * N_PAD, H), jnp.float32),
        mesh=mesh,
        compiler_params=pltpu.CompilerParams(use_tc_tiling_on_sc=False),
        scratch_types=[
            pltpu.VMEM((MSG_CHUNKS, CHUNK), jnp.int32),
            pltpu.VMEM((MSG_CHUNKS, CHUNK), jnp.int32),
            pltpu.VMEM((CHUNK, H), jnp.float32),
            pltpu.VMEM_SHARED((N_PAD, H), jnp.float32),
        ],
    )
    def sc_messages(h_hbm, src_hbm, dst_hbm, z_hbm, out_hbm,
                    src_v, dst_v, rows_v, acc_sh):
        c = lax.axis_index("c")
        s = lax.axis_index("s")
        wid = c * NS + s
        pltpu.sync_copy(z_hbm, acc_sh.at[pl.ds(s * ZROWS, ZROWS)])
        pltpu.sync_copy(src_hbm.at[wid], src_v)
        pltpu.sync_copy(dst_hbm.at[wid], dst_v)
        plsc.subcore_barrier()

        def body(j, carry):
            pltpu.sync_copy(h_hbm.at[src_v.at[j]], rows_v)
            pltpu.sync_copy(rows_v, acc_sh.at[dst_v.at[j]], add=True)
            return carry

        lax.fori_loop(0, MSG_CHUNKS, body, 0)
        plsc.subcore_barrier()
        pltpu.sync_copy(acc_sh.at[pl.ds(s * ZROWS, ZROWS)],
                        out_hbm.at[pl.ds(c * N_PAD + s * ZROWS, ZROWS)])

    @functools.partial(
        pl.kernel,
        out_type=[jax.ShapeDtypeStruct((EG_PAD, H), jnp.float32),
                  jax.ShapeDtypeStruct((EG_PAD, H), jnp.float32)],
        mesh=mesh,
        compiler_params=pltpu.CompilerParams(use_tc_tiling_on_sc=False),
        scratch_types=[
            pltpu.VMEM((EG_CHUNKS, CHUNK), jnp.int32),
            pltpu.VMEM((EG_CHUNKS, CHUNK), jnp.int32),
            pltpu.VMEM((CHUNK, H), jnp.float32),
            pltpu.VMEM((CHUNK, H), jnp.float32),
        ],
    )
    def sc_edge_gather(a_hbm, b_hbm, s_hbm, e_hbm, u_hbm, v_hbm,
                       s_v, e_v, ru, rv):
        c = lax.axis_index("c")
        s = lax.axis_index("s")
        wid = c * NS + s
        base = wid * EG_EPT
        pltpu.sync_copy(s_hbm.at[wid], s_v)
        pltpu.sync_copy(e_hbm.at[wid], e_v)

        def body(j, carry):
            off = base + j * CHUNK
            pltpu.sync_copy(a_hbm.at[s_v.at[j]], ru)
            pltpu.sync_copy(ru, u_hbm.at[pl.ds(off, CHUNK)])
            pltpu.sync_copy(b_hbm.at[e_v.at[j]], rv)
            pltpu.sync_copy(rv, v_hbm.at[pl.ds(off, CHUNK)])
            return carry

        lax.fori_loop(0, EG_CHUNKS, body, 0)

    return sc_messages, sc_edge_gather


def kernel(x, edge_index, enc_Ws, enc_bs, enc_gs, enc_betas,
           node_Ws, node_bs, node_gs, node_betas,
           edge_Ws, edge_bs, edge_gs, edge_betas):
    f32 = jnp.float32
    xp = jnp.pad(x.astype(f32), ((0, N_PAD - N), (0, 0)))
    start = edge_index[0].astype(jnp.int32)
    end = edge_index[1].astype(jnp.int32)

    src = jnp.concatenate([start, end])
    dst = jnp.concatenate([end, start])
    src = jnp.pad(src, (0, MSG_PAD - 2 * E), constant_values=DUMMY)
    dst = jnp.pad(dst, (0, MSG_PAD - 2 * E), constant_values=DUMMY)
    src = src.reshape(NW, MSG_CHUNKS, CHUNK)
    dst = dst.reshape(NW, MSG_CHUNKS, CHUNK)

    sg = jnp.pad(start, (0, EG_PAD - E)).reshape(NW, EG_CHUNKS, CHUNK)
    eg = jnp.pad(end, (0, EG_PAD - E)).reshape(NW, EG_CHUNKS, CHUNK)

    zrows = jnp.zeros((ZROWS, H), f32)
    r2 = lambda v: v.reshape(1, -1)
    sc_messages, sc_edge_gather = _sc_kernels()

    h = _enc(xp,
             enc_Ws[0], r2(enc_bs[0]), r2(enc_gs[0]), r2(enc_betas[0]),
             enc_Ws[1], r2(enc_bs[1]), r2(enc_gs[1]), r2(enc_betas[1]),
             enc_Ws[2], r2(enc_bs[2]), r2(enc_gs[2]), r2(enc_betas[2]))

    w1a, w1b = node_Ws[0][:H], node_Ws[0][H:]
    for _ in range(ITERS):
        parts = sc_messages(h, src, dst, zrows)
        h = _node(h, parts[:N_PAD], parts[N_PAD:],
                  w1a, w1b, r2(node_bs[0]), r2(node_gs[0]), r2(node_betas[0]),
                  node_Ws[1], r2(node_bs[1]), r2(node_gs[1]), r2(node_betas[1]),
                  node_Ws[2], r2(node_bs[2]), r2(node_gs[2]), r2(node_betas[2]))

    a, b = _ab(h, edge_Ws[0][:H], edge_Ws[0][H:])
    u, v = sc_edge_gather(a, b, sg, eg)
    out = _edge(u, v,
                r2(edge_bs[0]), r2(edge_gs[0]), r2(edge_betas[0]),
                edge_Ws[1], r2(edge_bs[1]), r2(edge_gs[1]), r2(edge_betas[1]),
                edge_Ws[2], r2(edge_bs[2]), r2(edge_gs[2]), r2(edge_betas[2]),
                edge_Ws[3], edge_bs[3].reshape(1, 1))
    return out[:E]


# speedup vs baseline: 3.7468x; 1.2388x over previous
import functools

import jax
import jax.numpy as jnp
from jax import lax
from jax.experimental import pallas as pl
from jax.experimental.pallas import tpu as pltpu
from jax.experimental.pallas import tpu_sc as plsc

N = 10000
E = 320000
H = 64
C_IN = 128
ITERS = 8
EPS = 1e-5

RB = 256
N_PAD = 10240
NBLK = N_PAD // RB

NC, NS = 2, 16
NW = NC * NS
CHUNK = 128

MSG_CHUNKS = 158
MSG_EPT = MSG_CHUNKS * CHUNK
MSG_PAD = NW * MSG_EPT

EG_CHUNKS = 80
EG_EPT = EG_CHUNKS * CHUNK
EG_PAD = NW * EG_EPT
EB = 512
EBLK = EG_PAD // EB

DUMMY = N_PAD - 1
ZROWS = N_PAD // NS


def _dot(a, b):
    return lax.dot_general(a.astype(jnp.bfloat16), b.astype(jnp.bfloat16),
                           (((1,), (0,)), ((), ())),
                           preferred_element_type=jnp.float32)


def _ln_relu(t, g, b):
    mu = jnp.mean(t, axis=-1, keepdims=True)
    var = jnp.mean((t - mu) ** 2, axis=-1, keepdims=True)
    t = (t - mu) * lax.rsqrt(var + EPS)
    return jnp.maximum(t * g + b, 0.0)


def _enc_body(x_ref, w1, b1, g1, be1, w2, b2, g2, be2, w3, b3, g3, be3, o_ref):
    t = _ln_relu(_dot(x_ref[...], w1[...]) + b1[...], g1[...], be1[...])
    t = _ln_relu(_dot(t, w2[...]) + b2[...], g2[...], be2[...])
    o_ref[...] = _ln_relu(_dot(t, w3[...]) + b3[...], g3[...], be3[...])


def _node_body(h_ref, m0_ref, m1_ref, w1a, w1b, b1, g1, be1,
               w2, b2, g2, be2, w3, b3, g3, be3, o_ref):
    m = m0_ref[...] + m1_ref[...]
    t = _dot(h_ref[...], w1a[...]) + _dot(m, w1b[...]) + b1[...]
    t = _ln_relu(t, g1[...], be1[...])
    t = _ln_relu(_dot(t, w2[...]) + b2[...], g2[...], be2[...])
    o_ref[...] = _ln_relu(_dot(t, w3[...]) + b3[...], g3[...], be3[...])


def _ab_body(h_ref, wa, wb, a_ref, b_ref):
    a_ref[...] = _dot(h_ref[...], wa[...])
    b_ref[...] = _dot(h_ref[...], wb[...])


def _edge_body(u_ref, v_ref, b1, g1, be1, w2, b2, g2, be2,
               w3, b3, g3, be3, w4, b4, o_ref):
    t = _ln_relu(u_ref[...] + v_ref[...] + b1[...], g1[...], be1[...])
    t = _ln_relu(_dot(t, w2[...]) + b2[...], g2[...], be2[...])
    t = _ln_relu(_dot(t, w3[...]) + b3[...], g3[...], be3[...])
    o_ref[...] = _dot(t, w4[...]) + b4[...]


def _wspec(r, c):
    return pl.BlockSpec((r, c), lambda i: (0, 0))


_enc = pl.pallas_call(
    _enc_body,
    grid=(NBLK,),
    in_specs=[pl.BlockSpec((RB, C_IN), lambda i: (i, 0)),
              _wspec(C_IN, H), _wspec(1, H), _wspec(1, H), _wspec(1, H),
              _wspec(H, H), _wspec(1, H), _wspec(1, H), _wspec(1, H),
              _wspec(H, H), _wspec(1, H), _wspec(1, H), _wspec(1, H)],
    out_specs=pl.BlockSpec((RB, H), lambda i: (i, 0)),
    out_shape=jax.ShapeDtypeStruct((N_PAD, H), jnp.float32),
)

_node = pl.pallas_call(
    _node_body,
    grid=(NBLK,),
    in_specs=[pl.BlockSpec((RB, H), lambda i: (i, 0)),
              pl.BlockSpec((RB, H), lambda i: (i, 0)),
              pl.BlockSpec((RB, H), lambda i: (i, 0)),
              _wspec(H, H), _wspec(H, H), _wspec(1, H), _wspec(1, H), _wspec(1, H),
              _wspec(H, H), _wspec(1, H), _wspec(1, H), _wspec(1, H),
              _wspec(H, H), _wspec(1, H), _wspec(1, H), _wspec(1, H)],
    out_specs=pl.BlockSpec((RB, H), lambda i: (i, 0)),
    out_shape=jax.ShapeDtypeStruct((N_PAD, H), jnp.float32),
)

_ab = pl.pallas_call(
    _ab_body,
    grid=(NBLK,),
    in_specs=[pl.BlockSpec((RB, H), lambda i: (i, 0)),
              _wspec(H, H), _wspec(H, H)],
    out_specs=[pl.BlockSpec((RB, H), lambda i: (i, 0)),
               pl.BlockSpec((RB, H), lambda i: (i, 0))],
    out_shape=[jax.ShapeDtypeStruct((N_PAD, H), jnp.float32),
               jax.ShapeDtypeStruct((N_PAD, H), jnp.float32)],
)

_edge = pl.pallas_call(
    _edge_body,
    grid=(EBLK,),
    in_specs=[pl.BlockSpec((EB, H), lambda i: (i, 0)),
              pl.BlockSpec((EB, H), lambda i: (i, 0)),
              _wspec(1, H), _wspec(1, H), _wspec(1, H),
              _wspec(H, H), _wspec(1, H), _wspec(1, H), _wspec(1, H),
              _wspec(H, H), _wspec(1, H), _wspec(1, H), _wspec(1, H),
              _wspec(H, 1), _wspec(1, 1)],
    out_specs=pl.BlockSpec((EB, 1), lambda i: (i, 0)),
    out_shape=jax.ShapeDtypeStruct((EG_PAD, 1), jnp.float32),
)


@functools.cache
def _sc_kernels():
    mesh = plsc.VectorSubcoreMesh(core_axis_name="c", subcore_axis_name="s",
                                  num_cores=NC, num_subcores=NS)

    @functools.partial(
        pl.kernel,
        out_type=jax.ShapeDtypeStruct((NC * N_PAD, H), jnp.float32),
        mesh=mesh,
        compiler_params=pltpu.CompilerParams(use_tc_tiling_on_sc=False),
        scratch_types=[
            pltpu.VMEM((MSG_CHUNKS, CHUNK), jnp.int32),
            pltpu.VMEM((MSG_CHUNKS, CHUNK), jnp.int32),
            pltpu.VMEM((CHUNK, H), jnp.float32),
            pltpu.VMEM((CHUNK, H), jnp.float32),
            pltpu.VMEM_SHARED((N_PAD, H), jnp.float32),
            pltpu.SemaphoreType.DMA,
            pltpu.SemaphoreType.DMA,
        ],
    )
    def sc_messages(h_hbm, src_hbm, dst_hbm, z_hbm, out_hbm,
                    src_v, dst_v, buf0, buf1, acc_sh, sem0, sem1):
        c = lax.axis_index("c")
        s = lax.axis_index("s")
        wid = c * NS + s
        pltpu.sync_copy(z_hbm, acc_sh.at[pl.ds(s * ZROWS, ZROWS)])
        pltpu.sync_copy(src_hbm.at[wid], src_v)
        pltpu.sync_copy(dst_hbm.at[wid], dst_v)
        plsc.subcore_barrier()

        pltpu.async_copy(h_hbm.at[src_v.at[0]], buf0, sem0)

        def body(i, carry):
            j0 = 2 * i
            j1 = 2 * i + 1
            jn = lax.rem(j1 + 1, MSG_CHUNKS)
            pltpu.async_copy(h_hbm.at[src_v.at[j1]], buf1, sem1)
            pltpu.make_async_copy(h_hbm.at[src_v.at[j0]], buf0, sem0).wait()
            pltpu.sync_copy(buf0, acc_sh.at[dst_v.at[j0]], add=True)
            pltpu.async_copy(h_hbm.at[src_v.at[jn]], buf0, sem0)
            pltpu.make_async_copy(h_hbm.at[src_v.at[j1]], buf1, sem1).wait()
            pltpu.sync_copy(buf1, acc_sh.at[dst_v.at[j1]], add=True)
            return carry

        lax.fori_loop(0, MSG_CHUNKS // 2, body, 0)
        pltpu.make_async_copy(h_hbm.at[src_v.at[0]], buf0, sem0).wait()
        plsc.subcore_barrier()
        pltpu.sync_copy(acc_sh.at[pl.ds(s * ZROWS, ZROWS)],
                        out_hbm.at[pl.ds(c * N_PAD + s * ZROWS, ZROWS)])

    @functools.partial(
        pl.kernel,
        out_type=[jax.ShapeDtypeStruct((EG_PAD, H), jnp.float32),
                  jax.ShapeDtypeStruct((EG_PAD, H), jnp.float32)],
        mesh=mesh,
        compiler_params=pltpu.CompilerParams(use_tc_tiling_on_sc=False),
        scratch_types=[
            pltpu.VMEM((EG_CHUNKS, CHUNK), jnp.int32),
            pltpu.VMEM((EG_CHUNKS, CHUNK), jnp.int32),
            pltpu.VMEM((CHUNK, H), jnp.float32),
            pltpu.VMEM((CHUNK, H), jnp.float32),
            pltpu.VMEM((CHUNK, H), jnp.float32),
            pltpu.VMEM((CHUNK, H), jnp.float32),
            pltpu.SemaphoreType.DMA,
            pltpu.SemaphoreType.DMA,
            pltpu.SemaphoreType.DMA,
            pltpu.SemaphoreType.DMA,
        ],
    )
    def sc_edge_gather(a_hbm, b_hbm, s_hbm, e_hbm, u_hbm, v_hbm,
                       s_v, e_v, a0, a1, b0, b1, sa0, sa1, sb0, sb1):
        c = lax.axis_index("c")
        s = lax.axis_index("s")
        wid = c * NS + s
        base = wid * EG_EPT
        pltpu.sync_copy(s_hbm.at[wid], s_v)
        pltpu.sync_copy(e_hbm.at[wid], e_v)

        pltpu.async_copy(a_hbm.at[s_v.at[0]], a0, sa0)
        pltpu.async_copy(b_hbm.at[e_v.at[0]], b0, sb0)

        def body(i, carry):
            j0 = 2 * i
            j1 = 2 * i + 1
            jn = lax.rem(j1 + 1, EG_CHUNKS)
            off0 = base + j0 * CHUNK
            off1 = base + j1 * CHUNK
            pltpu.async_copy(a_hbm.at[s_v.at[j1]], a1, sa1)
            pltpu.async_copy(b_hbm.at[e_v.at[j1]], b1, sb1)
            pltpu.make_async_copy(a_hbm.at[s_v.at[j0]], a0, sa0).wait()
            pltpu.sync_copy(a0, u_hbm.at[pl.ds(off0, CHUNK)])
            pltpu.make_async_copy(b_hbm.at[e_v.at[j0]], b0, sb0).wait()
            pltpu.sync_copy(b0, v_hbm.at[pl.ds(off0, CHUNK)])
            pltpu.async_copy(a_hbm.at[s_v.at[jn]], a0, sa0)
            pltpu.async_copy(b_hbm.at[e_v.at[jn]], b0, sb0)
            pltpu.make_async_copy(a_hbm.at[s_v.at[j1]], a1, sa1).wait()
            pltpu.sync_copy(a1, u_hbm.at[pl.ds(off1, CHUNK)])
            pltpu.make_async_copy(b_hbm.at[e_v.at[j1]], b1, sb1).wait()
            pltpu.sync_copy(b1, v_hbm.at[pl.ds(off1, CHUNK)])
            return carry

        lax.fori_loop(0, EG_CHUNKS // 2, body, 0)
        pltpu.make_async_copy(a_hbm.at[s_v.at[0]], a0, sa0).wait()
        pltpu.make_async_copy(b_hbm.at[e_v.at[0]], b0, sb0).wait()

    return sc_messages, sc_edge_gather


def kernel(x, edge_index, enc_Ws, enc_bs, enc_gs, enc_betas,
           node_Ws, node_bs, node_gs, node_betas,
           edge_Ws, edge_bs, edge_gs, edge_betas):
    f32 = jnp.float32
    xp = jnp.pad(x.astype(f32), ((0, N_PAD - N), (0, 0)))
    start = edge_index[0].astype(jnp.int32)
    end = edge_index[1].astype(jnp.int32)

    src = jnp.concatenate([start, end])
    dst = jnp.concatenate([end, start])
    src = jnp.pad(src, (0, MSG_PAD - 2 * E), constant_values=DUMMY)
    dst = jnp.pad(dst, (0, MSG_PAD - 2 * E), constant_values=DUMMY)
    src = src.reshape(NW, MSG_CHUNKS, CHUNK)
    dst = dst.reshape(NW, MSG_CHUNKS, CHUNK)

    sg = jnp.pad(start, (0, EG_PAD - E)).reshape(NW, EG_CHUNKS, CHUNK)
    eg = jnp.pad(end, (0, EG_PAD - E)).reshape(NW, EG_CHUNKS, CHUNK)

    zrows = jnp.zeros((ZROWS, H), f32)
    r2 = lambda v: v.reshape(1, -1)
    sc_messages, sc_edge_gather = _sc_kernels()

    h = _enc(xp,
             enc_Ws[0], r2(enc_bs[0]), r2(enc_gs[0]), r2(enc_betas[0]),
             enc_Ws[1], r2(enc_bs[1]), r2(enc_gs[1]), r2(enc_betas[1]),
             enc_Ws[2], r2(enc_bs[2]), r2(enc_gs[2]), r2(enc_betas[2]))

    w1a, w1b = node_Ws[0][:H], node_Ws[0][H:]
    for _ in range(ITERS):
        parts = sc_messages(h, src, dst, zrows)
        h = _node(h, parts[:N_PAD], parts[N_PAD:],
                  w1a, w1b, r2(node_bs[0]), r2(node_gs[0]), r2(node_betas[0]),
                  node_Ws[1], r2(node_bs[1]), r2(node_gs[1]), r2(node_betas[1]),
                  node_Ws[2], r2(node_bs[2]), r2(node_gs[2]), r2(node_betas[2]))

    a, b = _ab(h, edge_Ws[0][:H], edge_Ws[0][H:])
    u, v = sc_edge_gather(a, b, sg, eg)
    out = _edge(u, v,
                r2(edge_bs[0]), r2(edge_gs[0]), r2(edge_betas[0]),
                edge_Ws[1], r2(edge_bs[1]), r2(edge_gs[1]), r2(edge_betas[1]),
                edge_Ws[2], r2(edge_bs[2]), r2(edge_gs[2]), r2(edge_betas[2]),
                edge_Ws[3], edge_bs[3].reshape(1, 1))
    return out[:E]
